# baseline (speedup 1.0000x reference)
import functools

import jax
import jax.numpy as jnp
from jax import lax
from jax.experimental import pallas as pl
from jax.experimental.pallas import tpu as pltpu
from jax.experimental.pallas import tpu_sc as plsc

V_DIM = 1000000
EMB = 32
BATCH = 16384
CTX = 20

NC = 2
NS = 16
NW = NC * NS

TCOL_FULL = 7812
TAIL_LANES = V_DIM - TCOL_FULL * 128
K = 4
N_GROUPS = TCOL_FULL // K
GPW = N_GROUPS // NW
OUT_ROWS = V_DIM * EMB // 128


def _fire_in(tab, buf, sem, g):
    for r in range(4):
        for k in range(K):
            pltpu.async_copy(
                tab.at[pl.ds(8 * r, 8), pl.ds(512 * g + 128 * k, 128)],
                buf.at[4 * r + k],
                sem,
            )


def _drain_in(tab, buf, sem):
    for t in range(16):
        pltpu.make_async_copy(
            tab.at[pl.ds(0, 8), pl.ds(0, 128)], buf.at[t], sem
        ).wait()


def _drain_out(out, stag, sem):
    pltpu.make_async_copy(out.at[pl.ds(0, 16)], stag, sem).wait()


def _extract(in_ref, stag_ref, n_lq, ks):
    e16 = jnp.arange(16, dtype=jnp.int32)
    s_idx = e16 % 8
    tl = [(e16 // 8) * 4 + k for k in range(K)]
    th = [t + 8 for t in tl]

    def body(lq, carry):
        t_lo = lq // 8
        s_row = lq % 8
        for lr in range(4):
            lv = jnp.full((16,), 0, jnp.int32) + (4 * lq + lr)
            for k in ks:
                lo = plsc.load_gather(in_ref, [tl[k], s_idx, lv])
                hi = plsc.load_gather(in_ref, [th[k], s_idx, lv])
                stag_ref[4 * k + t_lo, s_row, pl.ds(32 * lr, 16)] = lo
                stag_ref[4 * k + t_lo, s_row, pl.ds(32 * lr + 16, 16)] = hi
        return carry

    lax.fori_loop(0, n_lq, body, 0)


def _detile_body(tab, tail, out, in0, in1, stag0, stag1, si0, si1, so0, so1):
    wid = lax.axis_index("s") * NC + lax.axis_index("c")
    g_base = wid * GPW

    def fire_out(stag, sem, g):
        pltpu.async_copy(stag, out.at[pl.ds(16 * g, 16)], sem)

    _fire_in(tab, in0, si0, g_base)

    @pl.loop(0, GPW // 2)
    def pair(i):
        g0 = g_base + 2 * i
        _fire_in(tab, in1, si1, g0 + 1)
        _drain_in(tab, in0, si0)

        @pl.when(i > 0)
        def _():
            _drain_out(out, stag0, so0)

        _extract(in0, stag0, 32, range(K))
        fire_out(stag0, so0, g0)
        _fire_in(tab, in0, si0, g0 + 2)
        _drain_in(tab, in1, si1)

        @pl.when(i > 0)
        def _():
            _drain_out(out, stag1, so1)

        _extract(in1, stag1, 32, range(K))
        fire_out(stag1, so1, g0 + 1)

    _drain_in(tab, in0, si0)
    _drain_out(out, stag0, so0)
    _extract(in0, stag0, 32, range(K))
    fire_out(stag0, so0, g_base + GPW - 1)
    _drain_out(out, stag1, so1)
    _drain_out(out, stag0, so0)

    @pl.when(wid == NW - 1)
    def _():
        _fire_in(tab, in0, si0, N_GROUPS - 1)
        _drain_in(tab, in0, si0)
        _extract(in0, stag0, 32, range(K))
        fire_out(stag0, so0, N_GROUPS - 1)
        _drain_out(out, stag0, so0)

    @pl.when(wid == NW - 2)
    def _():
        pltpu.sync_copy(tail, stag0.at[pl.ds(0, 2)])
        pltpu.sync_copy(stag0.at[pl.ds(0, 2)], out.at[pl.ds(16 * TCOL_FULL // 4, 2)])


BPW = BATCH // NW
IDX_PER_W = BPW * CTX
IDX_CHUNK = 128
ROWS_PER_STEP = 32
GATHERS_PER_STEP = ROWS_PER_STEP * CTX // IDX_CHUNK
N_STEPS = BPW // ROWS_PER_STEP
IDX_ROWS_PER_W = IDX_PER_W // IDX_CHUNK


def _tree_sum(vs):
    while len(vs) > 1:
        nxt = [vs[k] + vs[k + 1] for k in range(0, len(vs) - 1, 2)]
        if len(vs) % 2:
            nxt.append(vs[-1])
        vs = nxt
    return vs[0]


def _cbow_body(x_hbm, tab_hbm, out_hbm, idx_v, buf0, buf1, out_v, sem0, sem1):
    wid = lax.axis_index("s") * NC + lax.axis_index("c")

    pltpu.sync_copy(x_hbm.at[pl.ds(wid * IDX_ROWS_PER_W, IDX_ROWS_PER_W)], idx_v)

    bufs = (buf0, buf1)
    sems = (sem0, sem1)

    def fire(step, slot):
        cps = []
        for j in range(GATHERS_PER_STEP):
            cps.append(
                pltpu.async_copy(
                    tab_hbm.at[idx_v.at[step * GATHERS_PER_STEP + j]],
                    bufs[slot].at[pl.ds(j * IDX_CHUNK, IDX_CHUNK)],
                    sems[slot],
                )
            )
        return cps

    def reduce_step(step, slot):
        buf = bufs[slot]
        inv = jnp.float32(1.0 / CTX)

        def row_body(i, carry):
            base = i * CTX
            lo = [buf[base + j, 0:16] for j in range(CTX)]
            hi = [buf[base + j, 16:32] for j in range(CTX)]
            o = step * ROWS_PER_STEP + i
            out_v[o, 0:16] = _tree_sum(lo) * inv
            out_v[o, 16:32] = _tree_sum(hi) * inv
            return carry

        lax.fori_loop(0, ROWS_PER_STEP, row_body, 0)

    inflight = [fire(0, 0), fire(1, 1)]
    for g in range(N_STEPS):
        slot = g % 2
        for cp in inflight[slot]:
            cp.wait()
        reduce_step(g, slot)
        if g + 2 < N_STEPS:
            inflight[slot] = fire(g + 2, slot)

    pltpu.sync_copy(out_v, out_hbm.at[pl.ds(wid * BPW, BPW)])


@jax.jit
def _cbow(x2d, table):
    mesh = plsc.VectorSubcoreMesh(core_axis_name="c", subcore_axis_name="s")

    lin = pl.kernel(
        _detile_body,
        out_type=jax.ShapeDtypeStruct((OUT_ROWS // 8, 8, 128), jnp.float32),
        mesh=mesh,
        compiler_params=pltpu.CompilerParams(
            use_tc_tiling_on_sc=True, needs_layout_passes=False
        ),
        scratch_types=[
            pltpu.VMEM((16, 8, 128), jnp.float32),
            pltpu.VMEM((16, 8, 128), jnp.float32),
            pltpu.VMEM((16, 8, 128), jnp.float32),
            pltpu.VMEM((16, 8, 128), jnp.float32),
            pltpu.SemaphoreType.DMA,
            pltpu.SemaphoreType.DMA,
            pltpu.SemaphoreType.DMA,
            pltpu.SemaphoreType.DMA,
        ],
    )(table.T, table[128 * TCOL_FULL :].reshape(2, 8, 128))

    tbl = lin.reshape(V_DIM, EMB)

    return pl.kernel(
        _cbow_body,
        out_type=jax.ShapeDtypeStruct((BATCH, EMB), jnp.float32),
        mesh=mesh,
        compiler_params=pltpu.CompilerParams(use_tc_tiling_on_sc=False),
        scratch_types=[
            pltpu.VMEM((IDX_ROWS_PER_W, IDX_CHUNK), jnp.int32),
            pltpu.VMEM((ROWS_PER_STEP * CTX, EMB), jnp.float32),
            pltpu.VMEM((ROWS_PER_STEP * CTX, EMB), jnp.float32),
            pltpu.VMEM((BPW, EMB), jnp.float32),
            pltpu.SemaphoreType.DMA,
            pltpu.SemaphoreType.DMA,
        ],
    )(x2d, tbl)


def kernel(x, table):
    x2d = x.astype(jnp.int32).reshape(BATCH * CTX // IDX_CHUNK, IDX_CHUNK)
    return _cbow(x2d, table)


# speedup vs baseline: 1.0306x; 1.0306x over previous
import jax
import jax.numpy as jnp
from jax import lax
from jax.experimental import pallas as pl
from jax.experimental.pallas import tpu as pltpu
from jax.experimental.pallas import tpu_sc as plsc

V_DIM = 1000000
EMB = 32
BATCH = 16384
CTX = 20

NC = 2
NS = 16
NW = NC * NS

LANES_PER_BLK = 1024
N_BLKS = (V_DIM + LANES_PER_BLK - 1) // LANES_PER_BLK
OUT2_ROWS = N_BLKS * 256
V_PAD = OUT2_ROWS * 4


def _tc_transpose_body(i_ref, o_ref):
    y = i_ref[...].T
    o_ref[...] = jnp.concatenate(
        [y[0:256], y[256:512], y[512:768], y[768:1024]], axis=1
    )


BPW = BATCH // NW
IDX_PER_W = BPW * CTX
IDX_CHUNK = 128
ROWS_PER_STEP = 32
GATHERS_PER_STEP = ROWS_PER_STEP * CTX // IDX_CHUNK
N_STEPS = BPW // ROWS_PER_STEP
IDX_ROWS_PER_W = IDX_PER_W // IDX_CHUNK


def _tree_sum(vs):
    while len(vs) > 1:
        nxt = [vs[k] + vs[k + 1] for k in range(0, len(vs) - 1, 2)]
        if len(vs) % 2:
            nxt.append(vs[-1])
        vs = nxt
    return vs[0]


def _cbow_body(x_hbm, tab_hbm, out_hbm, idx_v, buf0, buf1, out_v, sem0, sem1):
    wid = lax.axis_index("s") * NC + lax.axis_index("c")

    pltpu.sync_copy(x_hbm.at[pl.ds(wid * IDX_ROWS_PER_W, IDX_ROWS_PER_W)], idx_v)

    bufs = (buf0, buf1)
    sems = (sem0, sem1)

    def permute_row(j):
        for cc in range(IDX_CHUNK // 16):
            v = idx_v[j, pl.ds(16 * cc, 16)]
            w = (v & -1024) | ((v & 255) << 2) | ((v >> 8) & 3)
            idx_v[j, pl.ds(16 * cc, 16)] = w

    def fire(step, slot):
        cps = []
        for j in range(GATHERS_PER_STEP):
            row = step * GATHERS_PER_STEP + j
            permute_row(row)
            cps.append(
                pltpu.async_copy(
                    tab_hbm.at[idx_v.at[row]],
                    bufs[slot].at[pl.ds(j * IDX_CHUNK, IDX_CHUNK)],
                    sems[slot],
                )
            )
        return cps

    def reduce_step(step, slot):
        buf = bufs[slot]
        inv = jnp.float32(1.0 / CTX)

        def row_body(i, carry):
            base = i * CTX
            lo = [buf[base + j, 0:16] for j in range(CTX)]
            hi = [buf[base + j, 16:32] for j in range(CTX)]
            o = step * ROWS_PER_STEP + i
            out_v[o, 0:16] = _tree_sum(lo) * inv
            out_v[o, 16:32] = _tree_sum(hi) * inv
            return carry

        lax.fori_loop(0, ROWS_PER_STEP, row_body, 0)

    inflight = [fire(0, 0), fire(1, 1)]
    for g in range(N_STEPS):
        slot = g % 2
        for cp in inflight[slot]:
            cp.wait()
        reduce_step(g, slot)
        if g + 2 < N_STEPS:
            inflight[slot] = fire(g + 2, slot)

    pltpu.sync_copy(out_v, out_hbm.at[pl.ds(wid * BPW, BPW)])


@jax.jit
def _cbow(x2d, table):
    lin = pl.pallas_call(
        _tc_transpose_body,
        grid=(N_BLKS,),
        in_specs=[pl.BlockSpec((32, LANES_PER_BLK), lambda i: (0, i))],
        out_specs=pl.BlockSpec((256, 128), lambda i: (i, 0)),
        out_shape=jax.ShapeDtypeStruct((OUT2_ROWS, 128), jnp.float32),
    )(table.T)

    tbl = lin.reshape(V_PAD, EMB)

    mesh = plsc.VectorSubcoreMesh(core_axis_name="c", subcore_axis_name="s")
    return pl.kernel(
        _cbow_body,
        out_type=jax.ShapeDtypeStruct((BATCH, EMB), jnp.float32),
        mesh=mesh,
        compiler_params=pltpu.CompilerParams(use_tc_tiling_on_sc=False),
        scratch_types=[
            pltpu.VMEM((IDX_ROWS_PER_W, IDX_CHUNK), jnp.int32),
            pltpu.VMEM((ROWS_PER_STEP * CTX, EMB), jnp.float32),
            pltpu.VMEM((ROWS_PER_STEP * CTX, EMB), jnp.float32),
            pltpu.VMEM((BPW, EMB), jnp.float32),
            pltpu.SemaphoreType.DMA,
            pltpu.SemaphoreType.DMA,
        ],
    )(x2d, tbl)


def kernel(x, table):
    x2d = x.astype(jnp.int32).reshape(BATCH * CTX // IDX_CHUNK, IDX_CHUNK)
    return _cbow(x2d, table)
